# baseline (speedup 1.0000x reference)
import functools

import jax
import jax.numpy as jnp
from jax import lax
from jax.experimental import pallas as pl
from jax.experimental.pallas import tpu as pltpu
from jax.experimental.pallas import tpu_sc as plsc

_V = 100000
_D = 300
_DP = 384
_B = 16384
_NC = 2
_NS = 16
_NW = _NC * _NS
_BPW = _B // _NW
_CHUNK = 128
_NCHUNK = _BPW // _CHUNK

_mesh = plsc.VectorSubcoreMesh(core_axis_name="c", subcore_axis_name="s")

_TCB = 2048
_NGB = (_V + _TCB - 1) // _TCB


def _tc_body(in_ref, out_ref):
    blk = in_ref[...]
    rows = jax.lax.broadcasted_iota(jnp.int32, (3 * _D, _D), 0)
    cols = jax.lax.broadcasted_iota(jnp.int32, (3 * _D, _D), 1)
    eye3 = jnp.where(rows % _D == cols, 1.0, 0.0).astype(jnp.float32)

    hi = blk.astype(jnp.bfloat16).astype(jnp.float32)
    r1 = blk - hi
    mid = r1.astype(jnp.bfloat16).astype(jnp.float32)
    lo = r1 - mid
    stacked = jnp.concatenate([hi, mid, lo], axis=0)

    out_ref[:, :_D] = jax.lax.dot_general(
        stacked, eye3, (((0,), (0,)), ((), ())),
        preferred_element_type=jnp.float32,
        precision=jax.lax.Precision.DEFAULT)


_tc_transpose = pl.pallas_call(
    _tc_body,
    grid=(_NGB,),
    in_specs=[pl.BlockSpec((_D, _TCB), lambda c: (0, c))],
    out_specs=pl.BlockSpec((_TCB, _DP), lambda c: (c, 0)),
    out_shape=jax.ShapeDtypeStruct((_V, _DP), jnp.float32),
)


@functools.partial(
    pl.kernel,
    mesh=_mesh,
    out_type=jax.ShapeDtypeStruct((_B, _DP), jnp.float32),
    scratch_types=[
        pltpu.VMEM((_NCHUNK, _CHUNK), jnp.int32),
        pltpu.VMEM((_CHUNK, _DP), jnp.float32),
        pltpu.VMEM((_CHUNK, _DP), jnp.float32),
        pltpu.SemaphoreType.DMA,
        pltpu.SemaphoreType.DMA,
        pltpu.SemaphoreType.DMA,
        pltpu.SemaphoreType.DMA,
    ],
)
def _gather_kernel(t2_hbm, idx_hbm, out_hbm, idx_v, rows0, rows1,
                   gsem0, gsem1, osem0, osem1):
    wid = lax.axis_index("s") * _NC + lax.axis_index("c")
    base = wid * _BPW

    pltpu.sync_copy(idx_hbm.at[wid], idx_v)

    bufs = (rows0, rows1)
    gsems = (gsem0, gsem1)
    osems = (osem0, osem1)

    def start_gather(c):
        return [pltpu.async_copy(
            t2_hbm.at[idx_v.at[c]], bufs[c % 2], gsems[c % 2])]

    gathers = [None] * _NCHUNK
    outs = [None] * _NCHUNK
    gathers[0] = start_gather(0)
    for c in range(_NCHUNK):
        nxt = c + 1
        if nxt < _NCHUNK:
            if nxt >= 2:
                outs[nxt - 2].wait()
                outs[nxt - 2] = None
            gathers[nxt] = start_gather(nxt)
        for cp in gathers[c]:
            cp.wait()
        outs[c] = pltpu.async_copy(
            bufs[c % 2], out_hbm.at[pl.ds(base + c * _CHUNK, _CHUNK)],
            osems[c % 2])
    for c in range(_NCHUNK):
        if outs[c] is not None:
            outs[c].wait()


def kernel(table, indices):
    tt = table.T
    idx = indices.astype(jnp.int32).reshape(_NW, _NCHUNK, _CHUNK)
    t2 = _tc_transpose(tt)
    out = _gather_kernel(t2, idx)
    return out[:, :_D]


# speedup vs baseline: 1.0708x; 1.0708x over previous
import functools

import jax
import jax.numpy as jnp
from jax import lax
from jax.experimental import pallas as pl
from jax.experimental.pallas import tpu as pltpu
from jax.experimental.pallas import tpu_sc as plsc

_V = 100000
_D = 300
_DP = 384
_B = 16384
_NC = 2
_NS = 16
_NW = _NC * _NS
_BPW = _B // _NW
_CHUNK = 128
_NCHUNK = _BPW // _CHUNK

_mesh = plsc.VectorSubcoreMesh(core_axis_name="c", subcore_axis_name="s")

_TCB = 4096
_NGB = (_V + _TCB - 1) // _TCB


def _tc_body(in_ref, out_ref):
    blk = in_ref[...]
    rows = jax.lax.broadcasted_iota(jnp.int32, (3 * _D, _D), 0)
    cols = jax.lax.broadcasted_iota(jnp.int32, (3 * _D, _D), 1)
    eye3 = jnp.where(rows % _D == cols, 1.0, 0.0).astype(jnp.float32)

    hi = blk.astype(jnp.bfloat16).astype(jnp.float32)
    r1 = blk - hi
    mid = r1.astype(jnp.bfloat16).astype(jnp.float32)
    lo = r1 - mid
    stacked = jnp.concatenate([hi, mid, lo], axis=0)

    out_ref[:, :_D] = jax.lax.dot_general(
        stacked, eye3, (((0,), (0,)), ((), ())),
        preferred_element_type=jnp.float32,
        precision=jax.lax.Precision.DEFAULT)


_tc_transpose = pl.pallas_call(
    _tc_body,
    grid=(_NGB,),
    in_specs=[pl.BlockSpec((_D, _TCB), lambda c: (0, c))],
    out_specs=pl.BlockSpec((_TCB, _DP), lambda c: (c, 0)),
    out_shape=jax.ShapeDtypeStruct((_V, _DP), jnp.float32),
)


@functools.partial(
    pl.kernel,
    mesh=_mesh,
    out_type=jax.ShapeDtypeStruct((_B, _DP), jnp.float32),
    scratch_types=[
        pltpu.VMEM((_NCHUNK, _CHUNK), jnp.int32),
        pltpu.VMEM((_CHUNK, _DP), jnp.float32),
        pltpu.VMEM((_CHUNK, _DP), jnp.float32),
        pltpu.SemaphoreType.DMA,
        pltpu.SemaphoreType.DMA,
        pltpu.SemaphoreType.DMA,
        pltpu.SemaphoreType.DMA,
    ],
)
def _gather_kernel(t2_hbm, idx_hbm, out_hbm, idx_v, rows0, rows1,
                   gsem0, gsem1, osem0, osem1):
    wid = lax.axis_index("s") * _NC + lax.axis_index("c")
    base = wid * _BPW

    pltpu.sync_copy(idx_hbm.at[wid], idx_v)

    bufs = (rows0, rows1)
    gsems = (gsem0, gsem1)
    osems = (osem0, osem1)

    def start_gather(c):
        return [pltpu.async_copy(
            t2_hbm.at[idx_v.at[c]], bufs[c % 2], gsems[c % 2])]

    gathers = [None] * _NCHUNK
    outs = [None] * _NCHUNK
    gathers[0] = start_gather(0)
    for c in range(_NCHUNK):
        nxt = c + 1
        if nxt < _NCHUNK:
            if nxt >= 2:
                outs[nxt - 2].wait()
                outs[nxt - 2] = None
            gathers[nxt] = start_gather(nxt)
        for cp in gathers[c]:
            cp.wait()
        outs[c] = pltpu.async_copy(
            bufs[c % 2], out_hbm.at[pl.ds(base + c * _CHUNK, _CHUNK)],
            osems[c % 2])
    for c in range(_NCHUNK):
        if outs[c] is not None:
            outs[c].wait()


def kernel(table, indices):
    tt = table.T
    idx = indices.astype(jnp.int32).reshape(_NW, _NCHUNK, _CHUNK)
    t2 = _tc_transpose(tt)
    out = _gather_kernel(t2, idx)
    return out[:, :_D]


# speedup vs baseline: 1.1429x; 1.0674x over previous
import functools

import jax
import jax.numpy as jnp
from jax import lax
from jax.experimental import pallas as pl
from jax.experimental.pallas import tpu as pltpu
from jax.experimental.pallas import tpu_sc as plsc

_V = 100000
_D = 300
_DA = 256
_DB = _D - _DA
_DP = 384
_B = 16384
_NC = 2
_NS = 16
_NW = _NC * _NS
_BPW = _B // _NW
_CHUNK = 128
_NCHUNK = _BPW // _CHUNK
_NSTRIP = _V // 128
_TAIL = _V - _NSTRIP * 128
_SPW = (_NSTRIP + _NW - 1) // _NW

_mesh = plsc.VectorSubcoreMesh(core_axis_name="c", subcore_axis_name="s")

_TCB = 4096
_NGB = (_V + _TCB - 1) // _TCB


def _tc_body(in_ref, out_ref):
    blk = in_ref[pl.ds(0, _DA), :]
    rows = jax.lax.broadcasted_iota(jnp.int32, (3 * _DA, _DA), 0)
    cols = jax.lax.broadcasted_iota(jnp.int32, (3 * _DA, _DA), 1)
    eye3 = jnp.where(rows % _DA == cols, 1.0, 0.0).astype(jnp.float32)

    hi = blk.astype(jnp.bfloat16).astype(jnp.float32)
    r1 = blk - hi
    mid = r1.astype(jnp.bfloat16).astype(jnp.float32)
    lo = r1 - mid
    stacked = jnp.concatenate([hi, mid, lo], axis=0)

    out_ref[...] = jax.lax.dot_general(
        stacked, eye3, (((0,), (0,)), ((), ())),
        preferred_element_type=jnp.float32,
        precision=jax.lax.Precision.DEFAULT)


_tc_transpose = pl.pallas_call(
    _tc_body,
    grid=(_NGB,),
    in_specs=[pl.BlockSpec((_D, _TCB), lambda c: (0, c))],
    out_specs=pl.BlockSpec((_TCB, _DA), lambda c: (c, 0)),
    out_shape=jax.ShapeDtypeStruct((_V, _DA), jnp.float32),
)


@functools.partial(
    pl.kernel,
    mesh=_mesh,
    compiler_params=pltpu.CompilerParams(needs_layout_passes=False),
    out_type=jax.ShapeDtypeStruct((_V, 128), jnp.float32),
    scratch_types=[
        pltpu.VMEM((_DB, 128), jnp.float32),
        pltpu.VMEM((_DB, 128), jnp.float32),
        pltpu.VMEM((128, 128), jnp.float32),
        pltpu.VMEM((128, 128), jnp.float32),
        pltpu.SemaphoreType.DMA,
        pltpu.SemaphoreType.DMA,
        pltpu.SemaphoreType.DMA,
        pltpu.SemaphoreType.DMA,
    ],
)
def _sc_transpose(tt_hbm, tail_hbm, t2b_hbm, inb0, inb1, ob0, ob1,
                  isem0, isem1, osem0, osem1):
    wid = lax.axis_index("s") * _NC + lax.axis_index("c")
    lanes = lax.iota(jnp.int32, 16)
    inbs = (inb0, inb1)
    isems = (isem0, isem1)
    obs = (ob0, ob1)
    osems = (osem0, osem1)

    def strip(k):
        ct = jnp.minimum(k * _NW + wid, _NSTRIP - 1)
        return pl.multiple_of(ct * 128, 128)

    def start_in(k):
        return pltpu.async_copy(
            tt_hbm.at[pl.ds(_DA, _DB), pl.ds(strip(k), 128)],
            inbs[k % 2], isems[k % 2])

    ins = [None] * _SPW
    outs = [None] * _SPW
    ins[0] = start_in(0)

    for k in range(_SPW):
        row0 = strip(k)
        ins[k].wait()
        if k + 1 < _SPW:
            ins[k + 1] = start_in(k + 1)
        inb = inbs[k % 2]
        if k >= 2:
            outs[k - 2].wait()
            outs[k - 2] = None

        @plsc.parallel_loop(0, _DB, unroll=4)
        def _(r):
            col = jnp.full((16,), r, jnp.int32)
            for g in range(8):
                vals = inb[r, pl.ds(g * 16, 16)]
                plsc.store_scatter(obs[k % 2], [g * 16 + lanes, col], vals)

        outs[k] = pltpu.async_copy(
            obs[k % 2], t2b_hbm.at[pl.ds(row0, 128)], osems[k % 2])

    for k in (_SPW - 2, _SPW - 1):
        if outs[k] is not None:
            outs[k].wait()

    @pl.when(wid == 0)
    def _():
        pltpu.sync_copy(tail_hbm, ob0.at[pl.ds(0, _TAIL)])
        pltpu.sync_copy(ob0.at[pl.ds(0, _TAIL)],
                        t2b_hbm.at[pl.ds(_NSTRIP * 128, _TAIL)])


@functools.partial(
    pl.kernel,
    mesh=_mesh,
    out_type=jax.ShapeDtypeStruct((_B, _DP), jnp.float32),
    scratch_types=[
        pltpu.VMEM((_NCHUNK, _CHUNK), jnp.int32),
        pltpu.VMEM((_CHUNK, _DP), jnp.float32),
        pltpu.VMEM((_CHUNK, _DP), jnp.float32),
        pltpu.SemaphoreType.DMA,
        pltpu.SemaphoreType.DMA,
        pltpu.SemaphoreType.DMA,
        pltpu.SemaphoreType.DMA,
    ],
)
def _gather_kernel(t2a_hbm, t2b_hbm, idx_hbm, out_hbm, idx_v, rows0, rows1,
                   gsem0, gsem1, osem0, osem1):
    wid = lax.axis_index("s") * _NC + lax.axis_index("c")
    base = wid * _BPW

    pltpu.sync_copy(idx_hbm.at[wid], idx_v)

    bufs = (rows0, rows1)
    gsems = (gsem0, gsem1)
    osems = (osem0, osem1)

    def start_gather(c):
        return [
            pltpu.async_copy(t2a_hbm.at[idx_v.at[c]],
                             bufs[c % 2].at[:, pl.ds(0, _DA)], gsems[c % 2]),
            pltpu.async_copy(t2b_hbm.at[idx_v.at[c]],
                             bufs[c % 2].at[:, pl.ds(_DA, 128)],
                             gsems[c % 2]),
        ]

    gathers = [None] * _NCHUNK
    outs = [None] * _NCHUNK
    gathers[0] = start_gather(0)
    for c in range(_NCHUNK):
        nxt = c + 1
        if nxt < _NCHUNK:
            if nxt >= 2:
                outs[nxt - 2].wait()
                outs[nxt - 2] = None
            gathers[nxt] = start_gather(nxt)
        for cp in gathers[c]:
            cp.wait()
        outs[c] = pltpu.async_copy(
            bufs[c % 2], out_hbm.at[pl.ds(base + c * _CHUNK, _CHUNK)],
            osems[c % 2])
    for c in range(_NCHUNK):
        if outs[c] is not None:
            outs[c].wait()


def kernel(table, indices):
    tt = table.T
    tail = jnp.pad(table[_NSTRIP * 128:, _DA:], ((0, 0), (0, 128 - _DB)))
    idx = indices.astype(jnp.int32).reshape(_NW, _NCHUNK, _CHUNK)
    t2b = _sc_transpose(tt, tail)
    t2a = _tc_transpose(tt)
    out = _gather_kernel(t2a, t2b, idx)
    return out[:, :_D]


# speedup vs baseline: 1.1833x; 1.0353x over previous
import functools

import jax
import jax.numpy as jnp
from jax import lax
from jax.experimental import pallas as pl
from jax.experimental.pallas import tpu as pltpu
from jax.experimental.pallas import tpu_sc as plsc

_V = 100000
_D = 300
_DA = 256
_DB = _D - _DA
_DP = 384
_B = 16384
_NC = 2
_NS = 16
_NW = _NC * _NS
_BPW = _B // _NW
_CHUNK = 128
_NCHUNK = _BPW // _CHUNK
_NSTRIP = _V // 128
_TAIL = _V - _NSTRIP * 128
_SPW = (_NSTRIP + _NW - 1) // _NW

_mesh = plsc.VectorSubcoreMesh(core_axis_name="c", subcore_axis_name="s")

_TCB = 4096
_NGB = (_V + _TCB - 1) // _TCB


def _tc_body(in_ref, out_ref):
    blk = in_ref[...]
    rows = jax.lax.broadcasted_iota(jnp.int32, (3 * _DA, _DA), 0)
    cols = jax.lax.broadcasted_iota(jnp.int32, (3 * _DA, _DA), 1)
    eye3 = jnp.where(rows % _DA == cols, 1.0, 0.0).astype(jnp.float32)

    hi = blk.astype(jnp.bfloat16).astype(jnp.float32)
    r1 = blk - hi
    mid = r1.astype(jnp.bfloat16).astype(jnp.float32)
    lo = r1 - mid
    stacked = jnp.concatenate([hi, mid, lo], axis=0)

    out_ref[...] = jax.lax.dot_general(
        stacked, eye3, (((0,), (0,)), ((), ())),
        preferred_element_type=jnp.float32,
        precision=jax.lax.Precision.DEFAULT)


_tc_transpose = pl.pallas_call(
    _tc_body,
    grid=(_NGB,),
    in_specs=[pl.BlockSpec((_DA, _TCB), lambda c: (0, c))],
    out_specs=pl.BlockSpec((_TCB, _DA), lambda c: (c, 0)),
    out_shape=jax.ShapeDtypeStruct((_V, _DA), jnp.float32),
)


@functools.partial(
    pl.kernel,
    mesh=_mesh,
    compiler_params=pltpu.CompilerParams(needs_layout_passes=False),
    out_type=jax.ShapeDtypeStruct((_V, 128), jnp.float32),
    scratch_types=[
        pltpu.VMEM((_DB, 128), jnp.float32),
        pltpu.VMEM((_DB, 128), jnp.float32),
        pltpu.VMEM((128, 128), jnp.float32),
        pltpu.VMEM((128, 128), jnp.float32),
        pltpu.SemaphoreType.DMA,
        pltpu.SemaphoreType.DMA,
        pltpu.SemaphoreType.DMA,
        pltpu.SemaphoreType.DMA,
    ],
)
def _sc_transpose(tt_hbm, tail_hbm, t2b_hbm, inb0, inb1, ob0, ob1,
                  isem0, isem1, osem0, osem1):
    wid = lax.axis_index("s") * _NC + lax.axis_index("c")
    lanes = lax.iota(jnp.int32, 16)
    inbs = (inb0, inb1)
    isems = (isem0, isem1)
    obs = (ob0, ob1)
    osems = (osem0, osem1)

    def strip(k):
        ct = jnp.minimum(k * _NW + wid, _NSTRIP - 1)
        return pl.multiple_of(ct * 128, 128)

    def start_in(k):
        return pltpu.async_copy(
            tt_hbm.at[pl.ds(_DA, _DB), pl.ds(strip(k), 128)],
            inbs[k % 2], isems[k % 2])

    ins = [None] * _SPW
    outs = [None] * _SPW
    ins[0] = start_in(0)

    for k in range(_SPW):
        row0 = strip(k)
        ins[k].wait()
        if k + 1 < _SPW:
            ins[k + 1] = start_in(k + 1)
        inb = inbs[k % 2]
        if k >= 2:
            outs[k - 2].wait()
            outs[k - 2] = None

        @plsc.parallel_loop(0, _DB, unroll=4)
        def _(r):
            col = jnp.full((16,), r, jnp.int32)
            for g in range(8):
                vals = inb[r, pl.ds(g * 16, 16)]
                plsc.store_scatter(obs[k % 2], [g * 16 + lanes, col], vals)

        outs[k] = pltpu.async_copy(
            obs[k % 2], t2b_hbm.at[pl.ds(row0, 128)], osems[k % 2])

    for k in (_SPW - 2, _SPW - 1):
        if outs[k] is not None:
            outs[k].wait()

    @pl.when(wid == 0)
    def _():
        pltpu.sync_copy(tail_hbm, ob0.at[pl.ds(0, _TAIL)])
        pltpu.sync_copy(ob0.at[pl.ds(0, _TAIL)],
                        t2b_hbm.at[pl.ds(_NSTRIP * 128, _TAIL)])


@functools.partial(
    pl.kernel,
    mesh=_mesh,
    out_type=jax.ShapeDtypeStruct((_B, _DP), jnp.float32),
    scratch_types=[
        pltpu.VMEM((_NCHUNK, _CHUNK), jnp.int32),
        pltpu.VMEM((_CHUNK, _DP), jnp.float32),
        pltpu.VMEM((_CHUNK, _DP), jnp.float32),
        pltpu.SemaphoreType.DMA,
        pltpu.SemaphoreType.DMA,
        pltpu.SemaphoreType.DMA,
        pltpu.SemaphoreType.DMA,
    ],
)
def _gather_kernel(t2a_hbm, t2b_hbm, idx_hbm, out_hbm, idx_v, rows0, rows1,
                   gsem0, gsem1, osem0, osem1):
    wid = lax.axis_index("s") * _NC + lax.axis_index("c")
    base = wid * _BPW

    pltpu.sync_copy(idx_hbm.at[wid], idx_v)

    bufs = (rows0, rows1)
    gsems = (gsem0, gsem1)
    osems = (osem0, osem1)

    def start_gather(c):
        return [
            pltpu.async_copy(t2a_hbm.at[idx_v.at[c]],
                             bufs[c % 2].at[:, pl.ds(0, _DA)], gsems[c % 2]),
            pltpu.async_copy(t2b_hbm.at[idx_v.at[c]],
                             bufs[c % 2].at[:, pl.ds(_DA, 128)],
                             gsems[c % 2]),
        ]

    gathers = [None] * _NCHUNK
    outs = [None] * _NCHUNK
    gathers[0] = start_gather(0)
    for c in range(_NCHUNK):
        nxt = c + 1
        if nxt < _NCHUNK:
            if nxt >= 2:
                outs[nxt - 2].wait()
                outs[nxt - 2] = None
            gathers[nxt] = start_gather(nxt)
        for cp in gathers[c]:
            cp.wait()
        outs[c] = pltpu.async_copy(
            bufs[c % 2], out_hbm.at[pl.ds(base + c * _CHUNK, _CHUNK)],
            osems[c % 2])
    for c in range(_NCHUNK):
        if outs[c] is not None:
            outs[c].wait()


def kernel(table, indices):
    tt = table.T
    tail = jnp.pad(table[_NSTRIP * 128:, _DA:], ((0, 0), (0, 128 - _DB)))
    idx = indices.astype(jnp.int32).reshape(_NW, _NCHUNK, _CHUNK)
    t2b = _sc_transpose(tt, tail)
    t2a = _tc_transpose(tt)
    out = _gather_kernel(t2a, t2b, idx)
    return out[:, :_D]


# speedup vs baseline: 1.1852x; 1.0017x over previous
import functools

import jax
import jax.numpy as jnp
from jax import lax
from jax.experimental import pallas as pl
from jax.experimental.pallas import tpu as pltpu
from jax.experimental.pallas import tpu_sc as plsc

_V = 100000
_D = 300
_DA = 256
_DB = _D - _DA
_DP = 384
_B = 16384
_NC = 2
_NS = 16
_NW = _NC * _NS
_BPW = _B // _NW
_CHUNK = 128
_NCHUNK = _BPW // _CHUNK
_NSTRIP = _V // 128
_TAIL = _V - _NSTRIP * 128
_SPW = (_NSTRIP + _NW - 1) // _NW

_mesh = plsc.VectorSubcoreMesh(core_axis_name="c", subcore_axis_name="s")

_TCB = 6144
_NGB = (_V + _TCB - 1) // _TCB


def _tc_body(in_ref, out_ref):
    blk = in_ref[...]
    rows = jax.lax.broadcasted_iota(jnp.int32, (3 * _DA, _DA), 0)
    cols = jax.lax.broadcasted_iota(jnp.int32, (3 * _DA, _DA), 1)
    eye3 = jnp.where(rows % _DA == cols, 1.0, 0.0).astype(jnp.float32)

    hi = blk.astype(jnp.bfloat16).astype(jnp.float32)
    r1 = blk - hi
    mid = r1.astype(jnp.bfloat16).astype(jnp.float32)
    lo = r1 - mid
    stacked = jnp.concatenate([hi, mid, lo], axis=0)

    out_ref[...] = jax.lax.dot_general(
        stacked, eye3, (((0,), (0,)), ((), ())),
        preferred_element_type=jnp.float32,
        precision=jax.lax.Precision.DEFAULT)


_tc_transpose = pl.pallas_call(
    _tc_body,
    grid=(_NGB,),
    in_specs=[pl.BlockSpec((_DA, _TCB), lambda c: (0, c))],
    out_specs=pl.BlockSpec((_TCB, _DA), lambda c: (c, 0)),
    out_shape=jax.ShapeDtypeStruct((_V, _DA), jnp.float32),
)


@functools.partial(
    pl.kernel,
    mesh=_mesh,
    compiler_params=pltpu.CompilerParams(needs_layout_passes=False),
    out_type=jax.ShapeDtypeStruct((_V, 128), jnp.float32),
    scratch_types=[
        pltpu.VMEM((_DB, 128), jnp.float32),
        pltpu.VMEM((_DB, 128), jnp.float32),
        pltpu.VMEM((128, 128), jnp.float32),
        pltpu.VMEM((128, 128), jnp.float32),
        pltpu.SemaphoreType.DMA,
        pltpu.SemaphoreType.DMA,
        pltpu.SemaphoreType.DMA,
        pltpu.SemaphoreType.DMA,
    ],
)
def _sc_transpose(tt_hbm, tail_hbm, t2b_hbm, inb0, inb1, ob0, ob1,
                  isem0, isem1, osem0, osem1):
    wid = lax.axis_index("s") * _NC + lax.axis_index("c")
    lanes = lax.iota(jnp.int32, 16)
    inbs = (inb0, inb1)
    isems = (isem0, isem1)
    obs = (ob0, ob1)
    osems = (osem0, osem1)

    def strip(k):
        ct = jnp.minimum(k * _NW + wid, _NSTRIP - 1)
        return pl.multiple_of(ct * 128, 128)

    def start_in(k):
        return pltpu.async_copy(
            tt_hbm.at[pl.ds(_DA, _DB), pl.ds(strip(k), 128)],
            inbs[k % 2], isems[k % 2])

    ins = [None] * _SPW
    outs = [None] * _SPW
    ins[0] = start_in(0)

    for k in range(_SPW):
        row0 = strip(k)
        ins[k].wait()
        if k + 1 < _SPW:
            ins[k + 1] = start_in(k + 1)
        inb = inbs[k % 2]
        if k >= 2:
            outs[k - 2].wait()
            outs[k - 2] = None

        @plsc.parallel_loop(0, _DB, unroll=4)
        def _(r):
            col = jnp.full((16,), r, jnp.int32)
            for g in range(8):
                vals = inb[r, pl.ds(g * 16, 16)]
                plsc.store_scatter(obs[k % 2], [g * 16 + lanes, col], vals)

        outs[k] = pltpu.async_copy(
            obs[k % 2], t2b_hbm.at[pl.ds(row0, 128)], osems[k % 2])

    for k in (_SPW - 2, _SPW - 1):
        if outs[k] is not None:
            outs[k].wait()

    @pl.when(wid == 0)
    def _():
        pltpu.sync_copy(tail_hbm, ob0.at[pl.ds(0, _TAIL)])
        pltpu.sync_copy(ob0.at[pl.ds(0, _TAIL)],
                        t2b_hbm.at[pl.ds(_NSTRIP * 128, _TAIL)])


@functools.partial(
    pl.kernel,
    mesh=_mesh,
    out_type=jax.ShapeDtypeStruct((_B, _DP), jnp.float32),
    scratch_types=[
        pltpu.VMEM((_NCHUNK, _CHUNK), jnp.int32),
        pltpu.VMEM((_CHUNK, _DP), jnp.float32),
        pltpu.VMEM((_CHUNK, _DP), jnp.float32),
        pltpu.SemaphoreType.DMA,
        pltpu.SemaphoreType.DMA,
        pltpu.SemaphoreType.DMA,
        pltpu.SemaphoreType.DMA,
    ],
)
def _gather_kernel(t2a_hbm, t2b_hbm, idx_hbm, out_hbm, idx_v, rows0, rows1,
                   gsem0, gsem1, osem0, osem1):
    wid = lax.axis_index("s") * _NC + lax.axis_index("c")
    base = wid * _BPW

    pltpu.sync_copy(idx_hbm.at[wid], idx_v)

    bufs = (rows0, rows1)
    gsems = (gsem0, gsem1)
    osems = (osem0, osem1)

    def start_gather(c):
        return [
            pltpu.async_copy(t2a_hbm.at[idx_v.at[c]],
                             bufs[c % 2].at[:, pl.ds(0, _DA)], gsems[c % 2]),
            pltpu.async_copy(t2b_hbm.at[idx_v.at[c]],
                             bufs[c % 2].at[:, pl.ds(_DA, 128)],
                             gsems[c % 2]),
        ]

    gathers = [None] * _NCHUNK
    outs = [None] * _NCHUNK
    gathers[0] = start_gather(0)
    for c in range(_NCHUNK):
        nxt = c + 1
        if nxt < _NCHUNK:
            if nxt >= 2:
                outs[nxt - 2].wait()
                outs[nxt - 2] = None
            gathers[nxt] = start_gather(nxt)
        for cp in gathers[c]:
            cp.wait()
        outs[c] = pltpu.async_copy(
            bufs[c % 2], out_hbm.at[pl.ds(base + c * _CHUNK, _CHUNK)],
            osems[c % 2])
    for c in range(_NCHUNK):
        if outs[c] is not None:
            outs[c].wait()


def kernel(table, indices):
    tt = table.T
    tail = jnp.pad(table[_NSTRIP * 128:, _DA:], ((0, 0), (0, 128 - _DB)))
    idx = indices.astype(jnp.int32).reshape(_NW, _NCHUNK, _CHUNK)
    t2b = _sc_transpose(tt, tail)
    t2a = _tc_transpose(tt)
    out = _gather_kernel(t2a, t2b, idx)
    return out[:, :_D]
